# Initial kernel scaffold; baseline (speedup 1.0000x reference)
import jax
import jax.numpy as jnp
from jax.experimental import pallas as pl


def kernel(input):
    raise NotImplementedError("write your pallas kernel here")


# speedup vs baseline: 3.5021x; 3.5021x over previous
import functools

import jax
import jax.numpy as jnp
import numpy as np
from jax import lax
from jax.experimental import pallas as pl
from jax.experimental.pallas import tpu as pltpu
from jax.experimental.pallas import tpu_sc as plsc

N_SHIFT = 2

B, C, T = 64, 128, 4096
NC, NS = 2, 16
NW = NC * NS
BPW = B // NW
R = 8
CHUNKS = C // R
L = 16
E = L


def _shift_constants():
    return (
        2, 2, -1, 2, 2, 2, 0, 0, 2, -1, 0, 2, -2, -1, 0, -2,
        1, 2, -2, 2, 0, 1, 1, 0, 2, -1, 0, -1, 0, 2, 2, 0,
        0, 1, -1, 2, 0, 2, 1, 1, 2, -1, -2, 2, -2, 0, -1, 2,
        0, 1, 1, -2, 0, 1, 2, 2, -1, -2, 0, -1, -2, -2, 2, -2,
    )


def _body(x_hbm, out_hbm, vin, vout):
    cid = lax.axis_index("c")
    sid = lax.axis_index("s")
    wid = sid * NC + cid

    shifts = _shift_constants()
    enc = jnp.int32((shifts[0] + N_SHIFT) * 5 + (shifts[1] + N_SHIFT))
    for w in range(1, NW):
        pw = (shifts[BPW * w] + N_SHIFT) * 5 + (shifts[BPW * w + 1] + N_SHIFT)
        enc = jnp.where(wid == w, jnp.int32(pw), enc)

    for j in range(BPW):
        b = wid * BPW + j
        s = (enc // 5 if j == 0 else enc % 5) - N_SHIFT

        d = L - s

        def chunk(k, _, b=b, d=d):
            r0 = k * R
            pltpu.sync_copy(x_hbm.at[b, pl.ds(r0, R), :], vin.at[:, pl.ds(L, T)])
            for r in range(R):
                vin[r, pl.ds(0, L)] = vin[r, pl.ds(T, L)]
                vin[r, pl.ds(T + L, L)] = vin[r, pl.ds(L, L)]

                def shuf(i, _, r=r):
                    vout[r, pl.ds(i * L, L)] = vin[r, pl.ds(i * L + d, L)]
                    return _

                lax.fori_loop(0, T // L, shuf, None)
            pltpu.sync_copy(vout, out_hbm.at[b, pl.ds(r0, R), :])
            return _

        lax.fori_loop(0, CHUNKS, chunk, None)


@jax.jit
def kernel(input):
    mesh = plsc.VectorSubcoreMesh(
        core_axis_name="c", subcore_axis_name="s", num_cores=NC, num_subcores=NS
    )
    f = pl.kernel(
        _body,
        out_type=jax.ShapeDtypeStruct((B, C, T), jnp.float32),
        mesh=mesh,
        compiler_params=pltpu.CompilerParams(use_tc_tiling_on_sc=False),
        scratch_types=[
            pltpu.VMEM((R, T + 2 * L), jnp.float32),
            pltpu.VMEM((R, T), jnp.float32),
        ],
    )
    return f(input)


# speedup vs baseline: 6.0669x; 1.7324x over previous
import functools

import jax
import jax.numpy as jnp
import numpy as np
from jax import lax
from jax.experimental import pallas as pl
from jax.experimental.pallas import tpu as pltpu
from jax.experimental.pallas import tpu_sc as plsc

N_SHIFT = 2

B, C, T = 64, 128, 4096
NC, NS = 2, 16
NW = NC * NS
BPW = B // NW
R = 8
CHUNKS = C // R
L = 16
E = L


def _shift_constants():
    return (
        2, 2, -1, 2, 2, 2, 0, 0, 2, -1, 0, 2, -2, -1, 0, -2,
        1, 2, -2, 2, 0, 1, 1, 0, 2, -1, 0, -1, 0, 2, 2, 0,
        0, 1, -1, 2, 0, 2, 1, 1, 2, -1, -2, 2, -2, 0, -1, 2,
        0, 1, 1, -2, 0, 1, 2, 2, -1, -2, 0, -1, -2, -2, 2, -2,
    )


def _body(x_hbm, out_hbm, vin, vout):
    cid = lax.axis_index("c")
    sid = lax.axis_index("s")
    wid = sid * NC + cid

    shifts = _shift_constants()
    enc = jnp.int32((shifts[0] + N_SHIFT) * 5 + (shifts[1] + N_SHIFT))
    for w in range(1, NW):
        pw = (shifts[BPW * w] + N_SHIFT) * 5 + (shifts[BPW * w + 1] + N_SHIFT)
        enc = jnp.where(wid == w, jnp.int32(pw), enc)

    for j in range(BPW):
        b = wid * BPW + j
        s = (enc // 5 if j == 0 else enc % 5) - N_SHIFT

        d = L - s

        def chunk(k, _, b=b, d=d):
            r0 = k * R
            pltpu.sync_copy(x_hbm.at[b, pl.ds(r0, R), :], vin.at[:, pl.ds(L, T)])
            for r in range(R):
                vin[r, pl.ds(0, L)] = vin[r, pl.ds(T, L)]
                vin[r, pl.ds(T + L, L)] = vin[r, pl.ds(L, L)]

                @plsc.parallel_loop(0, T // L, unroll=8)
                def shuf(i, r=r):
                    vout[r, pl.ds(i * L, L)] = vin[r, pl.ds(i * L + d, L)]
            pltpu.sync_copy(vout, out_hbm.at[b, pl.ds(r0, R), :])
            return _

        lax.fori_loop(0, CHUNKS, chunk, None)


@jax.jit
def kernel(input):
    mesh = plsc.VectorSubcoreMesh(
        core_axis_name="c", subcore_axis_name="s", num_cores=NC, num_subcores=NS
    )
    f = pl.kernel(
        _body,
        out_type=jax.ShapeDtypeStruct((B, C, T), jnp.float32),
        mesh=mesh,
        compiler_params=pltpu.CompilerParams(use_tc_tiling_on_sc=False),
        scratch_types=[
            pltpu.VMEM((R, T + 2 * L), jnp.float32),
            pltpu.VMEM((R, T), jnp.float32),
        ],
    )
    return f(input)


# speedup vs baseline: 7.0872x; 1.1682x over previous
import jax
import jax.numpy as jnp
from jax import lax
from jax.experimental import pallas as pl
from jax.experimental.pallas import tpu as pltpu
from jax.experimental.pallas import tpu_sc as plsc

N_SHIFT = 2

B, C, T = 64, 128, 4096
NC, NS = 2, 16
NW = NC * NS
BPW = B // NW
R = 4
CHPB = C // R
K = BPW * CHPB
L = 16


def _shift_constants():
    return (
        2, 2, -1, 2, 2, 2, 0, 0, 2, -1, 0, 2, -2, -1, 0, -2,
        1, 2, -2, 2, 0, 1, 1, 0, 2, -1, 0, -1, 0, 2, 2, 0,
        0, 1, -1, 2, 0, 2, 1, 1, 2, -1, -2, 2, -2, 0, -1, 2,
        0, 1, 1, -2, 0, 1, 2, 2, -1, -2, 0, -1, -2, -2, 2, -2,
    )


def _body(x_hbm, out_hbm, vin0, vin1, vout0, vout1, isem0, isem1, osem0, osem1):
    cid = lax.axis_index("c")
    sid = lax.axis_index("s")
    wid = sid * NC + cid

    shifts = _shift_constants()
    enc = jnp.int32((shifts[0] + N_SHIFT) * 5 + (shifts[1] + N_SHIFT))
    for w in range(1, NW):
        pw = (shifts[BPW * w] + N_SHIFT) * 5 + (shifts[BPW * w + 1] + N_SHIFT)
        enc = jnp.where(wid == w, jnp.int32(pw), enc)
    d0 = L - (enc // 5 - N_SHIFT)
    d1 = L - (enc % 5 - N_SHIFT)

    vin = (vin0, vin1)
    vout = (vout0, vout1)
    isem = (isem0, isem1)
    osem = (osem0, osem1)

    def in_slice(kk):
        b = wid * BPW + kk // CHPB
        r0 = (kk % CHPB) * R
        return x_hbm.at[b, pl.ds(r0, R), :]

    def out_slice(kk):
        b = wid * BPW + kk // CHPB
        r0 = (kk % CHPB) * R
        return out_hbm.at[b, pl.ds(r0, R), :]

    def start_in(kk, p):
        pltpu.make_async_copy(in_slice(kk), vin[p].at[:, pl.ds(L, T)], isem[p]).start()

    def shuffle(kk, p):
        d = jnp.where(kk < CHPB, d0, d1)
        pltpu.make_async_copy(in_slice(kk), vin[p].at[:, pl.ds(L, T)], isem[p]).wait()
        for r in range(R):
            vin[p][r, pl.ds(0, L)] = vin[p][r, pl.ds(T, L)]
            vin[p][r, pl.ds(T + L, L)] = vin[p][r, pl.ds(L, L)]

            @plsc.parallel_loop(0, T // L, unroll=8)
            def shuf(i, r=r, p=p, d=d):
                vout[p][r, pl.ds(i * L, L)] = vin[p][r, pl.ds(i * L + d, L)]

    def start_out(kk, p):
        pltpu.make_async_copy(vout[p], out_slice(kk), osem[p]).start()

    def wait_out(kk, p):
        pltpu.make_async_copy(vout[p], out_slice(kk), osem[p]).wait()

    start_in(0, 0)
    start_in(1, 1)

    def step(i, _):
        for p in range(2):
            kk = 2 * i + p
            shuffle(kk, p)
            start_in(kk + 2, p)
            start_out(kk, p)
        return _

    def step_drain(i, _):
        for p in range(2):
            kk = 2 * i + p
            wait_out(kk - 2, p)
            shuffle(kk, p)
            start_in(kk + 2, p)
            start_out(kk, p)
        return _

    step(0, None)
    lax.fori_loop(1, K // 2 - 1, step_drain, None)
    for p in range(2):
        kk = K - 2 + p
        wait_out(kk - 2, p)
        shuffle(kk, p)
        start_out(kk, p)
    for p in range(2):
        wait_out(K - 2 + p, p)


@jax.jit
def kernel(input):
    mesh = plsc.VectorSubcoreMesh(
        core_axis_name="c", subcore_axis_name="s", num_cores=NC, num_subcores=NS
    )
    f = pl.kernel(
        _body,
        out_type=jax.ShapeDtypeStruct((B, C, T), jnp.float32),
        mesh=mesh,
        compiler_params=pltpu.CompilerParams(use_tc_tiling_on_sc=False),
        scratch_types=[
            pltpu.VMEM((R, T + 2 * L), jnp.float32),
            pltpu.VMEM((R, T + 2 * L), jnp.float32),
            pltpu.VMEM((R, T), jnp.float32),
            pltpu.VMEM((R, T), jnp.float32),
            pltpu.SemaphoreType.DMA,
            pltpu.SemaphoreType.DMA,
            pltpu.SemaphoreType.DMA,
            pltpu.SemaphoreType.DMA,
        ],
    )
    return f(input)


# speedup vs baseline: 10.9083x; 1.5392x over previous
import jax
import jax.numpy as jnp
from jax import lax
from jax.experimental import pallas as pl
from jax.experimental.pallas import tpu as pltpu
from jax.experimental.pallas import tpu_sc as plsc

N_SHIFT = 2

B, C, T = 64, 128, 4096
SCB = 16
NC, NS = 2, 16
NW = NC * NS
R = 4
CHPB = C // R
K = SCB * CHPB // NW
L = 16


def _shift_constants():
    return (
        2, 2, -1, 2, 2, 2, 0, 0, 2, -1, 0, 2, -2, -1, 0, -2,
        1, 2, -2, 2, 0, 1, 1, 0, 2, -1, 0, -1, 0, 2, 2, 0,
        0, 1, -1, 2, 0, 2, 1, 1, 2, -1, -2, 2, -2, 0, -1, 2,
        0, 1, 1, -2, 0, 1, 2, 2, -1, -2, 0, -1, -2, -2, 2, -2,
    )


def _sc_body(x_hbm, out_hbm, vin0, vin1, vout0, vout1, isem0, isem1, osem0, osem1):
    cid = lax.axis_index("c")
    sid = lax.axis_index("s")
    wid = sid * NC + cid

    shifts = _shift_constants()
    d = jnp.int32(L - shifts[0])
    for w in range(1, NW):
        bw = w * K // CHPB
        d = jnp.where(wid == w, jnp.int32(L - shifts[bw]), d)

    vin = (vin0, vin1)
    vout = (vout0, vout1)
    isem = (isem0, isem1)
    osem = (osem0, osem1)

    def slices(kk):
        g = wid * K + kk
        b = g // CHPB
        r0 = (g % CHPB) * R
        return b, r0

    def start_in(kk, p):
        b, r0 = slices(kk)
        src = x_hbm.at[b, pl.ds(r0, R), :]
        pltpu.make_async_copy(src, vin[p].at[:, pl.ds(L, T)], isem[p]).start()

    def wait_in(kk, p):
        b, r0 = slices(kk)
        src = x_hbm.at[b, pl.ds(r0, R), :]
        pltpu.make_async_copy(src, vin[p].at[:, pl.ds(L, T)], isem[p]).wait()

    def shuffle(kk, p):
        wait_in(kk, p)
        for r in range(R):
            vin[p][r, pl.ds(0, L)] = vin[p][r, pl.ds(T, L)]
            vin[p][r, pl.ds(T + L, L)] = vin[p][r, pl.ds(L, L)]

            @plsc.parallel_loop(0, T // L, unroll=8)
            def shuf(i, r=r, p=p):
                vout[p][r, pl.ds(i * L, L)] = vin[p][r, pl.ds(i * L + d, L)]

    def start_out(kk, p):
        b, r0 = slices(kk)
        pltpu.make_async_copy(vout[p], out_hbm.at[b, pl.ds(r0, R), :], osem[p]).start()

    def wait_out(kk, p):
        b, r0 = slices(kk)
        pltpu.make_async_copy(vout[p], out_hbm.at[b, pl.ds(r0, R), :], osem[p]).wait()

    start_in(0, 0)
    start_in(1, 1)

    for p in range(2):
        shuffle(p, p)
        start_in(p + 2, p)
        start_out(p, p)

    def step_drain(i, _):
        for p in range(2):
            kk = 2 * i + p
            wait_out(kk - 2, p)
            shuffle(kk, p)
            start_in(kk + 2, p)
            start_out(kk, p)
        return _

    lax.fori_loop(1, K // 2 - 1, step_drain, None)

    for p in range(2):
        kk = K - 2 + p
        wait_out(kk - 2, p)
        shuffle(kk, p)
        start_out(kk, p)
    for p in range(2):
        wait_out(K - 2 + p, p)


def _sc_kernel(input):
    mesh = plsc.VectorSubcoreMesh(
        core_axis_name="c", subcore_axis_name="s", num_cores=NC, num_subcores=NS
    )
    f = pl.kernel(
        _sc_body,
        out_type=jax.ShapeDtypeStruct((SCB, C, T), jnp.float32),
        mesh=mesh,
        compiler_params=pltpu.CompilerParams(use_tc_tiling_on_sc=False),
        scratch_types=[
            pltpu.VMEM((R, T + 2 * L), jnp.float32),
            pltpu.VMEM((R, T + 2 * L), jnp.float32),
            pltpu.VMEM((R, T), jnp.float32),
            pltpu.VMEM((R, T), jnp.float32),
            pltpu.SemaphoreType.DMA,
            pltpu.SemaphoreType.DMA,
            pltpu.SemaphoreType.DMA,
            pltpu.SemaphoreType.DMA,
        ],
    )
    return f(input)


def _tc_roll_body(x_ref, o_ref):
    shifts = _shift_constants()
    b = pl.program_id(0) + SCB
    s = jnp.int32(shifts[SCB])
    for w in range(SCB + 1, B):
        s = jnp.where(b == w, jnp.int32(shifts[w]), s)
    o_ref[0] = pltpu.roll(x_ref[0], s, axis=1)


def _tc_kernel(input):
    return pl.pallas_call(
        _tc_roll_body,
        grid=(B - SCB,),
        in_specs=[pl.BlockSpec((1, C, T), lambda i: (i + SCB, 0, 0))],
        out_specs=pl.BlockSpec((1, C, T), lambda i: (i + SCB, 0, 0)),
        out_shape=jax.ShapeDtypeStruct((B, C, T), jnp.float32),
    )(input)


@jax.jit
def kernel(input):
    sc_out = _sc_kernel(input[:SCB])
    tc_out = _tc_kernel(input)
    return lax.dynamic_update_slice(tc_out, sc_out, (0, 0, 0))


# speedup vs baseline: 17.4514x; 1.5998x over previous
import jax
import jax.numpy as jnp
from jax import lax
from jax.experimental import pallas as pl
from jax.experimental.pallas import tpu as pltpu
from jax.experimental.pallas import tpu_sc as plsc

N_SHIFT = 2

B, C, T = 64, 128, 4096
SCB = 16
NC, NS = 2, 16
NW = NC * NS
R = 8
CHPB = C // R
K = SCB * CHPB // NW
L = 16
LW = T + 2 * L


def _shift_constants():
    return (
        2, 2, -1, 2, 2, 2, 0, 0, 2, -1, 0, 2, -2, -1, 0, -2,
        1, 2, -2, 2, 0, 1, 1, 0, 2, -1, 0, -1, 0, 2, 2, 0,
        0, 1, -1, 2, 0, 2, 1, 1, 2, -1, -2, 2, -2, 0, -1, 2,
        0, 1, 1, -2, 0, 1, 2, 2, -1, -2, 0, -1, -2, -2, 2, -2,
    )


def _sc_body(x_hbm, out_hbm, vin0, vin1, lin, isem0, isem1, osem0, osem1):
    cid = lax.axis_index("c")
    sid = lax.axis_index("s")
    wid = sid * NC + cid

    shifts = _shift_constants()
    d = jnp.int32(L - shifts[0])
    for w in range(1, NW):
        bw = w * K // CHPB
        d = jnp.where(wid == w, jnp.int32(L - shifts[bw]), d)

    vin = (vin0, vin1)
    isem = (isem0, isem1)
    osem = (osem0, osem1)

    def slices(kk):
        g = wid * K + kk
        b = g // CHPB
        r0 = (g % CHPB) * R
        return b, r0

    def start_in(kk, p):
        b, r0 = slices(kk)
        pltpu.make_async_copy(x_hbm.at[b, pl.ds(r0, R), :], vin[p], isem[p]).start()

    def wait_in(kk, p):
        b, r0 = slices(kk)
        pltpu.make_async_copy(x_hbm.at[b, pl.ds(r0, R), :], vin[p], isem[p]).wait()

    def start_out(kk, p):
        b, r0 = slices(kk)
        pltpu.make_async_copy(vin[p], out_hbm.at[b, pl.ds(r0, R), :], osem[p]).start()

    def wait_out(kk, p):
        b, r0 = slices(kk)
        pltpu.make_async_copy(vin[p], out_hbm.at[b, pl.ds(r0, R), :], osem[p]).wait()

    def pass1(p):
        for r in range(R):
            base = r * LW

            @plsc.parallel_loop(0, T // L, unroll=8)
            def detile(i, r=r, p=p, base=base):
                lin[pl.ds(base + L + i * L, L)] = vin[p][r, pl.ds(i * L, L)]

            lin[pl.ds(base, L)] = vin[p][r, pl.ds(T - L, L)]
            lin[pl.ds(base + L + T, L)] = vin[p][r, pl.ds(0, L)]

    def pass2(p):
        for r in range(R):
            base = r * LW

            @plsc.parallel_loop(0, T // L, unroll=8)
            def shuf(i, r=r, p=p, base=base):
                vin[p][r, pl.ds(i * L, L)] = lin[pl.ds(base + i * L + d, L)]

    start_in(0, 0)
    start_in(1, 1)

    def step(kk, _):
        p = lax.rem(kk, 2)

        def proc(p):
            wait_in(kk, p)
            pass1(p)
            @pl.when(jnp.logical_and(kk >= 1, kk + 1 < K))
            def _():
                wait_out(kk - 1, 1 - p)
                start_in(kk + 1, 1 - p)

            pass2(p)
            start_out(kk, p)

        @pl.when(p == 0)
        def _():
            proc(0)

        @pl.when(p == 1)
        def _():
            proc(1)

        return _

    lax.fori_loop(0, K, step, None)

    wait_out(K - 2, (K - 2) % 2)
    wait_out(K - 1, (K - 1) % 2)


def _sc_kernel(input):
    mesh = plsc.VectorSubcoreMesh(
        core_axis_name="c", subcore_axis_name="s", num_cores=NC, num_subcores=NS
    )
    f = pl.kernel(
        _sc_body,
        out_type=jax.ShapeDtypeStruct((SCB, C, T), jnp.float32),
        mesh=mesh,
        scratch_types=[
            pltpu.VMEM((R, T), jnp.float32),
            pltpu.VMEM((R, T), jnp.float32),
            pltpu.VMEM((R * LW,), jnp.float32),
            pltpu.SemaphoreType.DMA,
            pltpu.SemaphoreType.DMA,
            pltpu.SemaphoreType.DMA,
            pltpu.SemaphoreType.DMA,
        ],
    )
    return f(input)


def _tc_roll_body(x_ref, o_ref):
    shifts = _shift_constants()
    b = pl.program_id(0) + SCB
    s = jnp.int32(shifts[SCB])
    for w in range(SCB + 1, B):
        s = jnp.where(b == w, jnp.int32(shifts[w]), s)
    o_ref[0] = pltpu.roll(x_ref[0], s, axis=1)


def _tc_kernel(input):
    return pl.pallas_call(
        _tc_roll_body,
        grid=(B - SCB,),
        in_specs=[pl.BlockSpec((1, C, T), lambda i: (i + SCB, 0, 0))],
        out_specs=pl.BlockSpec((1, C, T), lambda i: (i + SCB, 0, 0)),
        out_shape=jax.ShapeDtypeStruct((B, C, T), jnp.float32),
    )(input)


@jax.jit
def kernel(input):
    sc_out = _sc_kernel(input)
    tc_out = _tc_kernel(input)
    return lax.dynamic_update_slice(tc_out, sc_out, (0, 0, 0))
